# baseline (speedup 1.0000x reference)
import functools

import jax
import jax.numpy as jnp
from jax import lax
from jax.experimental import pallas as pl
from jax.experimental.pallas import tpu as pltpu
from jax.experimental.pallas import tpu_sc as plsc

_N = 10000
_E = 320000
_V = 2
_D = 128
_W = _V * _D
_NT = 16
_CH = 64
_G = 16
_NG = 20
_NCH = _G * _NG
_EPAD = _NT * _NCH * _CH
_HALF = 5120
_NDUM = 256
_ACC = _HALF + _NDUM
_RPT = _ACC // _NT
_CPO = 16


def _zero_acc(rows, acc, s):
    def zr(i, _):
        for v in range(_V):
            for k in range(_D // 16):
                rows[i, v, pl.ds(k * 16, 16)] = jnp.zeros((16,), jnp.float32)
        return 0
    lax.fori_loop(0, _CPO, zr, 0)
    for k in range(_RPT // _CPO):
        pltpu.sync_copy(rows.at[pl.ds(0, _CPO)],
                        acc.at[pl.ds(s * _RPT + k * _CPO, _CPO)])


def _spmm_phase(tbl, srcm, dstm, out, idx_s, idx_d, rows, acc,
                gsem, ssem, isem, c, s):
    _zero_acc(rows.at[0], acc, s)
    plsc.subcore_barrier()

    pltpu.async_copy(srcm.at[s, 0], idx_s.at[pl.ds(0, _G)], isem.at[0])
    pltpu.async_copy(dstm.at[c, s, 0], idx_d.at[pl.ds(0, _G)], isem.at[0])
    pltpu.async_copy(srcm.at[s, 1], idx_s.at[pl.ds(_G, _G)], isem.at[1])
    pltpu.async_copy(dstm.at[c, s, 1], idx_d.at[pl.ds(_G, _G)], isem.at[1])

    def _one_group(slot, g):
        sb = slot * _G
        pltpu.make_async_copy(srcm.at[s, g], idx_s.at[pl.ds(sb, _G)],
                              isem.at[slot]).wait()
        pltpu.make_async_copy(dstm.at[c, s, g], idx_d.at[pl.ds(sb, _G)],
                              isem.at[slot]).wait()
        gh = [None] * _G
        sh = [None] * _G
        gh[0] = pltpu.async_copy(tbl.at[idx_s.at[sb + 0]], rows.at[0], gsem)
        for jj in range(_G):
            b = jj % 2
            gh[jj].wait()
            if jj + 1 < _G:
                if jj >= 1:
                    sh[jj - 1].wait()
                gh[jj + 1] = pltpu.async_copy(
                    tbl.at[idx_s.at[sb + jj + 1]], rows.at[1 - b], gsem)
            sh[jj] = pltpu.async_copy(
                rows.at[b], acc.at[idx_d.at[sb + jj]], ssem, add=True)
        sh[_G - 2].wait()
        sh[_G - 1].wait()
        @pl.when(g + 2 < _NG)
        def _():
            pltpu.async_copy(srcm.at[s, g + 2], idx_s.at[pl.ds(sb, _G)],
                             isem.at[slot])
            pltpu.async_copy(dstm.at[c, s, g + 2], idx_d.at[pl.ds(sb, _G)],
                             isem.at[slot])

    def grp2(h, _):
        _one_group(0, 2 * h)
        _one_group(1, 2 * h + 1)
        return 0
    lax.fori_loop(0, _NG // 2, grp2, 0)
    plsc.subcore_barrier()
    for k in range(_RPT // _CPO):
        base = s * _RPT + k * _CPO
        pltpu.sync_copy(acc.at[pl.ds(base, _CPO)], rows.at[0, pl.ds(0, _CPO)])
        pltpu.sync_copy(rows.at[0, pl.ds(0, _CPO)],
                        out.at[c, pl.ds(base, _CPO)])
    plsc.subcore_barrier()


def _mesh():
    return plsc.VectorSubcoreMesh(core_axis_name="c", subcore_axis_name="s")


def _scratch():
    return [
        pltpu.VMEM((2 * _G, _CH), jnp.int32),
        pltpu.VMEM((2 * _G, _CH), jnp.int32),
        pltpu.VMEM((2, _CH, _V, _D), jnp.float32),
        pltpu.VMEM_SHARED((_ACC, _V, _D), jnp.float32),
        pltpu.SemaphoreType.DMA,
        pltpu.SemaphoreType.DMA,
        pltpu.SemaphoreType.DMA((2,)),
    ]


def _spmm_pair(tblA, srcA, dstA, tblB, srcB, dstB):
    o = jax.ShapeDtypeStruct((2, _ACC, _V, _D), jnp.float32)

    @functools.partial(pl.kernel, mesh=_mesh(), out_type=(o, o),
                       scratch_types=_scratch())
    def k(tA, sA, dA, tB, sB, dB, out1, out2, idx_s, idx_d, rows, acc,
          gsem, ssem, isem):
        c = lax.axis_index("c")
        s = lax.axis_index("s")
        _spmm_phase(tA, sA, dA, out1, idx_s, idx_d, rows, acc,
                    gsem, ssem, isem, c, s)
        _spmm_phase(tB, sB, dB, out2, idx_s, idx_d, rows, acc,
                    gsem, ssem, isem, c, s)

    return k(tblA, srcA, dstA, tblB, srcB, dstB)


def _spmm_single(tbl, src, dst):
    o = jax.ShapeDtypeStruct((2, _ACC, _V, _D), jnp.float32)

    @functools.partial(pl.kernel, mesh=_mesh(), out_type=o,
                       scratch_types=_scratch())
    def k(t, sm, dm, out, idx_s, idx_d, rows, acc, gsem, ssem, isem):
        c = lax.axis_index("c")
        s = lax.axis_index("s")
        _spmm_phase(t, sm, dm, out, idx_s, idx_d, rows, acc,
                    gsem, ssem, isem, c, s)

    return k(tbl, src, dst)


_BN = 1000


def _l2norm_body(x_ref, o_ref):
    x = x_ref[...]
    a = x[:, :_D]
    b = x[:, _D:]
    na = jnp.sqrt(jnp.sum(a * a, axis=-1, keepdims=True))
    nb = jnp.sqrt(jnp.sum(b * b, axis=-1, keepdims=True))
    o_ref[...] = jnp.concatenate(
        [a / jnp.maximum(na, 1e-12), b / jnp.maximum(nb, 1e-12)], axis=1)


def _l2norm_tc(x):
    r = x.shape[0]
    return pl.pallas_call(
        _l2norm_body,
        grid=(r // 1024,),
        in_specs=[pl.BlockSpec((1024, _W), lambda i: (i, 0))],
        out_specs=pl.BlockSpec((1024, _W), lambda i: (i, 0)),
        out_shape=jax.ShapeDtypeStruct((r, _W), jnp.float32),
    )(x)


def _ln(x, g, b):
    m = jnp.mean(x, axis=-1, keepdims=True)
    d = x - m
    v = jnp.mean(d * d, axis=-1, keepdims=True)
    return d * jax.lax.rsqrt(v + 1e-5) * g + b


def _epilogue_body(ss1_ref, ss2_ref, w1t_ref, b1_ref, g1_ref, be1_ref,
                   w2t_ref, b2_ref, g2_ref, be2_ref, inwt_ref, inb_ref,
                   outwt_ref, outb_ref, lng_ref, lnb_ref, mf_ref, o_ref):
    f32 = jnp.float32
    mf = mf_ref[...]
    scale = 1.0 / jnp.sqrt(jnp.float32(_D // 4))
    outs = []
    for v in range(_V):
        x1 = ss1_ref[:, v * _D:(v + 1) * _D]
        x2 = ss2_ref[:, v * _D:(v + 1) * _D]
        n1 = jnp.sqrt(jnp.sum(x1 * x1, axis=-1, keepdims=True))
        x1 = x1 / jnp.maximum(n1, 1e-12)
        n2 = jnp.sqrt(jnp.sum(x2 * x2, axis=-1, keepdims=True))
        x2 = x2 / jnp.maximum(n2, 1e-12)
        h1 = jnp.maximum(
            _ln(jnp.dot(x1, w1t_ref[...], preferred_element_type=f32)
                + b1_ref[...], g1_ref[...], be1_ref[...]), 0.0)
        h2 = jnp.maximum(
            _ln(jnp.dot(x2, w2t_ref[...], preferred_element_type=f32)
                + b2_ref[...], g2_ref[...], be2_ref[...]), 0.0)
        qkv1 = jnp.dot(h1, inwt_ref[...], preferred_element_type=f32) \
            + inb_ref[...]
        qkv2 = jnp.dot(h2, inwt_ref[...], preferred_element_type=f32) \
            + inb_ref[...]
        q1, k1, v1 = qkv1[:, :_D], qkv1[:, _D:2 * _D], qkv1[:, 2 * _D:]
        q2, k2, v2 = qkv2[:, :_D], qkv2[:, _D:2 * _D], qkv2[:, 2 * _D:]
        s11 = jnp.dot(q1 * k1, mf, preferred_element_type=f32) * scale
        s12 = jnp.dot(q1 * k2, mf, preferred_element_type=f32) * scale
        s21 = jnp.dot(q2 * k1, mf, preferred_element_type=f32) * scale
        s22 = jnp.dot(q2 * k2, mf, preferred_element_type=f32) * scale
        m1 = jnp.maximum(s11, s12)
        e11 = jnp.exp(s11 - m1)
        e12 = jnp.exp(s12 - m1)
        r1 = 1.0 / (e11 + e12)
        o1 = (e11 * r1) * v1 + (e12 * r1) * v2
        m2 = jnp.maximum(s21, s22)
        e21 = jnp.exp(s21 - m2)
        e22 = jnp.exp(s22 - m2)
        r2 = 1.0 / (e21 + e22)
        o2 = (e21 * r2) * v1 + (e22 * r2) * v2
        a1 = jnp.dot(o1, outwt_ref[...], preferred_element_type=f32) \
            + outb_ref[...]
        a2 = jnp.dot(o2, outwt_ref[...], preferred_element_type=f32) \
            + outb_ref[...]
        t1 = _ln(a1 + h1, lng_ref[...], lnb_ref[...])
        t2 = _ln(a2 + h2, lng_ref[...], lnb_ref[...])
        outs.append(0.5 * (t1 + t2))
    o_ref[...] = jnp.stack(outs, axis=1)


def _epilogue_tc(ss1, ss2, w1t, b1, g1, be1, w2t, b2, g2, be2,
                 inwt, inb, outwt, outb, lng, lnb, mf):
    def seg(i):
        return (i, 0)

    def full(i):
        return (0, 0)

    return pl.pallas_call(
        _epilogue_body,
        grid=(_N // _BN,),
        in_specs=[
            pl.BlockSpec((_BN, _W), seg),
            pl.BlockSpec((_BN, _W), seg),
            pl.BlockSpec((_D, _D), full),
            pl.BlockSpec((1, _D), full),
            pl.BlockSpec((1, _D), full),
            pl.BlockSpec((1, _D), full),
            pl.BlockSpec((_D, _D), full),
            pl.BlockSpec((1, _D), full),
            pl.BlockSpec((1, _D), full),
            pl.BlockSpec((1, _D), full),
            pl.BlockSpec((_D, 3 * _D), full),
            pl.BlockSpec((1, 3 * _D), full),
            pl.BlockSpec((_D, _D), full),
            pl.BlockSpec((1, _D), full),
            pl.BlockSpec((1, _D), full),
            pl.BlockSpec((1, _D), full),
            pl.BlockSpec((_D, _D), full),
        ],
        out_specs=pl.BlockSpec((_BN, _V, _D), lambda i: (i, 0, 0)),
        out_shape=jax.ShapeDtypeStruct((_N, _V, _D), jnp.float32),
    )(ss1, ss2, w1t, b1, g1, be1, w2t, b2, g2, be2,
      inwt, inb, outwt, outb, lng, lnb, mf)


def _prep_edges(edge):
    pad = _EPAD - _E
    src = jnp.concatenate([edge[0], jnp.zeros((pad,), jnp.int32)])
    dst = jnp.concatenate([edge[1], jnp.zeros((pad,), jnp.int32)])
    valid = jnp.arange(_EPAD, dtype=jnp.int32) < _E
    dummy = _HALF + (jnp.arange(_EPAD, dtype=jnp.int32) % _NDUM)
    dst_lo = jnp.where(valid & (dst < _HALF), dst, dummy)
    dst_hi = jnp.where(valid & (dst >= _HALF), dst - _HALF, dummy)
    src = src.reshape(_NT, _NG, _G, _CH)
    dst2 = jnp.stack([dst_lo, dst_hi]).reshape(2, _NT, _NG, _G, _CH)
    return src, dst2


def _halves(o):
    return jnp.concatenate([o[0, :_HALF], o[1, :_HALF]],
                           axis=0).reshape(2 * _HALF, _W)


def kernel(feat_A, feat_P, edge_AP, edge_PA, W1, b1, g1, beta1, W2, b2, g2,
           beta2, attn_in_w, attn_in_b, attn_out_w, attn_out_b, ln_g, ln_b):
    srcAP, dstAP = _prep_edges(edge_AP)
    srcPA, dstPA = _prep_edges(edge_PA)
    tbl1 = feat_A
    tbl2a = feat_P

    out1, out2a = _spmm_pair(tbl1, srcAP, dstAP, tbl2a, srcPA, dstPA)
    tbl2 = _l2norm_tc(_halves(out2a))
    out2 = _spmm_single(tbl2.reshape(2 * _HALF, _V, _D), srcAP, dstAP)

    ids = jnp.arange(_D, dtype=jnp.int32) // (_D // 4)
    mf = (ids[:, None] == ids[None, :]).astype(jnp.float32)
    r2 = lambda x: x.reshape(1, -1)
    h_P = _epilogue_tc(_halves(out1), _halves(out2),
                       W1.T, r2(b1), r2(g1), r2(beta1),
                       W2.T, r2(b2), r2(g2), r2(beta2),
                       attn_in_w.T, r2(attn_in_b), attn_out_w.T,
                       r2(attn_out_b), r2(ln_g), r2(ln_b), mf)
    return feat_A, h_P


# speedup vs baseline: 1.6896x; 1.6896x over previous
import functools

import jax
import jax.numpy as jnp
from jax import lax
from jax.experimental import pallas as pl
from jax.experimental.pallas import tpu as pltpu
from jax.experimental.pallas import tpu_sc as plsc

_N = 10000
_E = 320000
_V = 2
_D = 128
_NT = 16
_CH = 64
_NS = 4
_G = 16
_NG = 20
_NCH = _G * _NG
_EPAD = _NT * _NCH * _CH
_ACC = 10240
_RPT = _ACC // _NT
_DUMMY = _N


def _zero_acc(rows, acc, s):
    def zr(i, _):
        for k in range(8):
            rows[i, pl.ds(k * 16, 16)] = jnp.zeros((16,), jnp.float32)
        return 0
    lax.fori_loop(0, _CH, zr, 0)
    for k in range(_RPT // _CH):
        pltpu.sync_copy(rows, acc.at[pl.ds(s * _RPT + k * _CH, _CH)])


def _spmm_phase(tbl, srcm, dstm, out, idx_s, idx_d, rows, acc, gsem, ssem,
                isem, c, s):
    _zero_acc(rows.at[0], acc, s)
    plsc.subcore_barrier()

    pltpu.async_copy(srcm.at[c, s, 0], idx_s.at[0], isem.at[0])
    pltpu.async_copy(dstm.at[s, 0], idx_d.at[0], isem.at[0])
    pltpu.async_copy(srcm.at[c, s, 1], idx_s.at[1], isem.at[1])
    pltpu.async_copy(dstm.at[s, 1], idx_d.at[1], isem.at[1])

    def _one_group(slot, g):
        pltpu.make_async_copy(srcm.at[c, s, g], idx_s.at[slot],
                              isem.at[slot]).wait()
        pltpu.make_async_copy(dstm.at[s, g], idx_d.at[slot],
                              isem.at[slot]).wait()
        gh = [None] * _G
        sh = [None] * _G
        for p in range(_NS - 1):
            gh[p] = pltpu.async_copy(tbl.at[idx_s.at[slot, p]], rows.at[p],
                                     gsem)
        for jj in range(_G):
            b = jj % _NS
            gh[jj].wait()
            nxt = jj + _NS - 1
            if nxt < _G:
                if jj >= 1:
                    sh[jj - 1].wait()
                gh[nxt] = pltpu.async_copy(
                    tbl.at[idx_s.at[slot, nxt]], rows.at[nxt % _NS], gsem)
            sh[jj] = pltpu.async_copy(
                rows.at[b], acc.at[idx_d.at[slot, jj]], ssem, add=True)
        for jj in range(max(0, _G - _NS), _G):
            sh[jj].wait()
        @pl.when(g + 2 < _NG)
        def _():
            pltpu.async_copy(srcm.at[c, s, g + 2], idx_s.at[slot],
                             isem.at[slot])
            pltpu.async_copy(dstm.at[s, g + 2], idx_d.at[slot],
                             isem.at[slot])

    def grp2(h, _):
        _one_group(0, 2 * h)
        _one_group(1, 2 * h + 1)
        return 0
    lax.fori_loop(0, _NG // 2, grp2, 0)
    plsc.subcore_barrier()
    for k in range(_RPT // _CH):
        base = s * _RPT + k * _CH
        pltpu.sync_copy(acc.at[pl.ds(base, _CH)], rows.at[0])
        pltpu.sync_copy(rows.at[0], out.at[c, pl.ds(base, _CH)])
    plsc.subcore_barrier()


def _spmm_pair(tblA, srcA, dstA, tblB, srcB, dstB):
    mesh = plsc.VectorSubcoreMesh(core_axis_name="c", subcore_axis_name="s")
    o = jax.ShapeDtypeStruct((_V, _ACC, _D), jnp.float32)

    @functools.partial(
        pl.kernel, mesh=mesh, out_type=(o, o),
        scratch_types=[
            pltpu.VMEM((2, _G, _CH), jnp.int32),
            pltpu.VMEM((2, _G, _CH), jnp.int32),
            pltpu.VMEM((_NS, _CH, _D), jnp.float32),
            pltpu.VMEM_SHARED((_ACC, _D), jnp.float32),
            pltpu.SemaphoreType.DMA,
            pltpu.SemaphoreType.DMA,
            pltpu.SemaphoreType.DMA((2,)),
        ],
    )
    def k(tA, sA, dA, tB, sB, dB, out1, out2, idx_s, idx_d, rows, acc,
          gsem, ssem, isem):
        c = lax.axis_index("c")
        s = lax.axis_index("s")
        _spmm_phase(tA, sA, dA, out1, idx_s, idx_d, rows, acc, gsem, ssem,
                    isem, c, s)
        _spmm_phase(tB, sB, dB, out2, idx_s, idx_d, rows, acc, gsem, ssem,
                    isem, c, s)

    return k(tblA, srcA, dstA, tblB, srcB, dstB)


def _spmm_single(tbl, src, dst):
    mesh = plsc.VectorSubcoreMesh(core_axis_name="c", subcore_axis_name="s")
    o = jax.ShapeDtypeStruct((_V, _ACC, _D), jnp.float32)

    @functools.partial(
        pl.kernel, mesh=mesh, out_type=o,
        scratch_types=[
            pltpu.VMEM((2, _G, _CH), jnp.int32),
            pltpu.VMEM((2, _G, _CH), jnp.int32),
            pltpu.VMEM((_NS, _CH, _D), jnp.float32),
            pltpu.VMEM_SHARED((_ACC, _D), jnp.float32),
            pltpu.SemaphoreType.DMA,
            pltpu.SemaphoreType.DMA,
            pltpu.SemaphoreType.DMA((2,)),
        ],
    )
    def k(t, sm, dm, out, idx_s, idx_d, rows, acc, gsem, ssem, isem):
        c = lax.axis_index("c")
        s = lax.axis_index("s")
        _spmm_phase(t, sm, dm, out, idx_s, idx_d, rows, acc, gsem, ssem,
                    isem, c, s)

    return k(tbl, src, dst)


_BN = 1000


def _l2norm_body(x_ref, o_ref):
    x = x_ref[...]
    n = jnp.sqrt(jnp.sum(x * x, axis=-1, keepdims=True))
    o_ref[...] = x / jnp.maximum(n, 1e-12)


def _l2norm_tc(x):
    return pl.pallas_call(
        _l2norm_body,
        grid=(_N // _BN, _V),
        in_specs=[pl.BlockSpec((1, _BN, _D), lambda i, v: (v, i, 0))],
        out_specs=pl.BlockSpec((1, _BN, _D), lambda i, v: (v, i, 0)),
        out_shape=jax.ShapeDtypeStruct((_V, _N, _D), jnp.float32),
    )(x)


def _ln(x, g, b):
    m = jnp.mean(x, axis=-1, keepdims=True)
    d = x - m
    v = jnp.mean(d * d, axis=-1, keepdims=True)
    return d * jax.lax.rsqrt(v + 1e-5) * g + b


def _epilogue_body(ss1_ref, ss2_ref, w1t_ref, b1_ref, g1_ref, be1_ref,
                   w2t_ref, b2_ref, g2_ref, be2_ref, inwt_ref, inb_ref,
                   outwt_ref, outb_ref, lng_ref, lnb_ref, mf_ref, o_ref):
    f32 = jnp.float32
    x1 = ss1_ref[0]
    x2 = ss2_ref[0]
    n1 = jnp.sqrt(jnp.sum(x1 * x1, axis=-1, keepdims=True))
    x1 = x1 / jnp.maximum(n1, 1e-12)
    n2 = jnp.sqrt(jnp.sum(x2 * x2, axis=-1, keepdims=True))
    x2 = x2 / jnp.maximum(n2, 1e-12)
    h1 = jnp.maximum(_ln(jnp.dot(x1, w1t_ref[...], preferred_element_type=f32)
                         + b1_ref[...], g1_ref[...], be1_ref[...]), 0.0)
    h2 = jnp.maximum(_ln(jnp.dot(x2, w2t_ref[...], preferred_element_type=f32)
                         + b2_ref[...], g2_ref[...], be2_ref[...]), 0.0)
    qkv1 = jnp.dot(h1, inwt_ref[...], preferred_element_type=f32) + inb_ref[...]
    qkv2 = jnp.dot(h2, inwt_ref[...], preferred_element_type=f32) + inb_ref[...]
    q1, k1, v1 = qkv1[:, :_D], qkv1[:, _D:2 * _D], qkv1[:, 2 * _D:]
    q2, k2, v2 = qkv2[:, :_D], qkv2[:, _D:2 * _D], qkv2[:, 2 * _D:]
    mf = mf_ref[...]
    scale = 1.0 / jnp.sqrt(jnp.float32(_D // 4))
    s11 = jnp.dot(q1 * k1, mf, preferred_element_type=f32) * scale
    s12 = jnp.dot(q1 * k2, mf, preferred_element_type=f32) * scale
    s21 = jnp.dot(q2 * k1, mf, preferred_element_type=f32) * scale
    s22 = jnp.dot(q2 * k2, mf, preferred_element_type=f32) * scale
    m1 = jnp.maximum(s11, s12)
    e11 = jnp.exp(s11 - m1)
    e12 = jnp.exp(s12 - m1)
    r1 = 1.0 / (e11 + e12)
    o1 = (e11 * r1) * v1 + (e12 * r1) * v2
    m2 = jnp.maximum(s21, s22)
    e21 = jnp.exp(s21 - m2)
    e22 = jnp.exp(s22 - m2)
    r2 = 1.0 / (e21 + e22)
    o2 = (e21 * r2) * v1 + (e22 * r2) * v2
    a1 = jnp.dot(o1, outwt_ref[...], preferred_element_type=f32) + outb_ref[...]
    a2 = jnp.dot(o2, outwt_ref[...], preferred_element_type=f32) + outb_ref[...]
    t1 = _ln(a1 + h1, lng_ref[...], lnb_ref[...])
    t2 = _ln(a2 + h2, lng_ref[...], lnb_ref[...])
    o_ref[...] = (0.5 * (t1 + t2))[None, :, :]


def _epilogue_tc(ss1, ss2, w1t, b1, g1, be1, w2t, b2, g2, be2,
                 inwt, inb, outwt, outb, lng, lnb, mf):
    def seg(i, v):
        return (v, i, 0)

    def full(i, v):
        return (0, 0)

    return pl.pallas_call(
        _epilogue_body,
        grid=(_N // _BN, _V),
        in_specs=[
            pl.BlockSpec((1, _BN, _D), seg),
            pl.BlockSpec((1, _BN, _D), seg),
            pl.BlockSpec((_D, _D), full),
            pl.BlockSpec((1, _D), full),
            pl.BlockSpec((1, _D), full),
            pl.BlockSpec((1, _D), full),
            pl.BlockSpec((_D, _D), full),
            pl.BlockSpec((1, _D), full),
            pl.BlockSpec((1, _D), full),
            pl.BlockSpec((1, _D), full),
            pl.BlockSpec((_D, 3 * _D), full),
            pl.BlockSpec((1, 3 * _D), full),
            pl.BlockSpec((_D, _D), full),
            pl.BlockSpec((1, _D), full),
            pl.BlockSpec((1, _D), full),
            pl.BlockSpec((1, _D), full),
            pl.BlockSpec((_D, _D), full),
        ],
        out_specs=pl.BlockSpec((1, _BN, _D), seg),
        out_shape=jax.ShapeDtypeStruct((_V, _N, _D), jnp.float32),
    )(ss1, ss2, w1t, b1, g1, be1, w2t, b2, g2, be2,
      inwt, inb, outwt, outb, lng, lnb, mf)


def _prep_edges(edge):
    pad = _EPAD - _E
    src = jnp.concatenate([edge[0], jnp.zeros((pad,), jnp.int32)])
    dst = jnp.concatenate([edge[1], jnp.full((pad,), _DUMMY, jnp.int32)])
    src = src.reshape(_NT, _NG, _G, _CH)
    dst = dst.reshape(_NT, _NG, _G, _CH)
    src2 = jnp.stack([src, src + _N])
    return src2, dst


def kernel(feat_A, feat_P, edge_AP, edge_PA, W1, b1, g1, beta1, W2, b2, g2,
           beta2, attn_in_w, attn_in_b, attn_out_w, attn_out_b, ln_g, ln_b):
    srcAP, dstAP = _prep_edges(edge_AP)
    srcPA, dstPA = _prep_edges(edge_PA)
    tblA = feat_A.transpose(1, 0, 2).reshape(_V * _N, _D)
    tblP = feat_P.transpose(1, 0, 2).reshape(_V * _N, _D)

    seg1, seg2a = _spmm_pair(tblA, srcAP, dstAP, tblP, srcPA, dstPA)
    h2a = _l2norm_tc(seg2a)
    seg2 = _spmm_single(h2a.reshape(_V * _N, _D), srcAP, dstAP)

    ids = jnp.arange(_D, dtype=jnp.int32) // (_D // 4)
    mf = (ids[:, None] == ids[None, :]).astype(jnp.float32)
    r2 = lambda x: x.reshape(1, -1)
    h_P = _epilogue_tc(seg1, seg2, W1.T, r2(b1), r2(g1), r2(beta1),
                       W2.T, r2(b2), r2(g2), r2(beta2),
                       attn_in_w.T, r2(attn_in_b), attn_out_w.T,
                       r2(attn_out_b), r2(ln_g), r2(ln_b), mf)
    return feat_A, h_P.transpose(1, 0, 2)
